# baseline (speedup 1.0000x reference)
import functools

import jax
import jax.numpy as jnp
from jax import lax
from jax.experimental import pallas as pl
from jax.experimental.pallas import tpu as pltpu, tpu_sc as plsc

ROLE_CAD, ROLE_EMB = 13, 8
RACE_CAD, RACE_EMB = 5, 4
GEND_CAD, GEND_EMB = 3, 2
ALIGN_CAD, ALIGN_EMB = 3, 2
B = 16384
D = ROLE_EMB + RACE_EMB + GEND_EMB + ALIGN_EMB

NC, NS, L = 2, 16, 16
NW = NC * NS
BPW = B // NW
CHUNKS = BPW // L

OFF_ROLE = 0
OFF_RACE = OFF_ROLE + ROLE_CAD * ROLE_EMB
OFF_GEND = OFF_RACE + RACE_CAD * RACE_EMB
OFF_ALIGN = OFF_GEND + GEND_CAD * GEND_EMB
FLAT_LEN = OFF_ALIGN + ALIGN_CAD * ALIGN_EMB


def _hero_body(role_h, race_h, gend_h, align_h, tab_h, out_h,
               flat_v, ri_v, ci_v, gi_v, ai_v, out_v):
    wid = lax.axis_index("s") * NC + lax.axis_index("c")
    base = wid * BPW

    pltpu.sync_copy(tab_h, flat_v)
    pltpu.sync_copy(role_h.at[pl.ds(base, BPW)], ri_v)
    pltpu.sync_copy(race_h.at[pl.ds(base, BPW)], ci_v)
    pltpu.sync_copy(gend_h.at[pl.ds(base, BPW)], gi_v)
    pltpu.sync_copy(align_h.at[pl.ds(base, BPW)], ai_v)

    lane = lax.iota(jnp.int32, L)
    col_vecs = [jnp.full((L,), col, jnp.int32) for col in range(D)]

    @plsc.parallel_loop(0, CHUNKS, step=1, unroll=4)
    def chunk_body(k):
        e0 = k * L
        rows = e0 + lane
        r = ri_v[pl.ds(e0, L)]
        c = ci_v[pl.ds(e0, L)]
        g = gi_v[pl.ds(e0, L)]
        a = ai_v[pl.ds(e0, L)]
        a = jnp.minimum(jnp.maximum(a + 1, 0), ALIGN_CAD - 1)
        for col in range(D):
            if col < ROLE_EMB:
                off = OFF_ROLE + r * ROLE_EMB + col
            elif col < ROLE_EMB + RACE_EMB:
                off = OFF_RACE + c * RACE_EMB + (col - ROLE_EMB)
            elif col < ROLE_EMB + RACE_EMB + GEND_EMB:
                off = OFF_GEND + g * GEND_EMB + (col - ROLE_EMB - RACE_EMB)
            else:
                off = (OFF_ALIGN + a * ALIGN_EMB
                       + (col - ROLE_EMB - RACE_EMB - GEND_EMB))
            vals = plsc.load_gather(flat_v, [off])
            plsc.store_scatter(out_v, [rows, col_vecs[col]], vals)

    pltpu.sync_copy(out_v, out_h.at[pl.ds(base, BPW)])


_hero = functools.partial(
    pl.kernel,
    out_type=jax.ShapeDtypeStruct((B, D), jnp.float32),
    mesh=plsc.VectorSubcoreMesh(core_axis_name="c", subcore_axis_name="s"),
    compiler_params=pltpu.CompilerParams(needs_layout_passes=False),
    scratch_types=[
        pltpu.VMEM((FLAT_LEN,), jnp.float32),
        pltpu.VMEM((BPW,), jnp.int32),
        pltpu.VMEM((BPW,), jnp.int32),
        pltpu.VMEM((BPW,), jnp.int32),
        pltpu.VMEM((BPW,), jnp.int32),
        pltpu.VMEM((BPW, D), jnp.float32),
    ],
)(_hero_body)


def kernel(role, race, gend, align, role_table, race_table, gend_table,
           align_table):
    tables = jnp.zeros((FLAT_LEN,), jnp.float32)
    tables = lax.dynamic_update_slice(tables, role_table.reshape(-1),
                                      (OFF_ROLE,))
    tables = lax.dynamic_update_slice(tables, race_table.reshape(-1),
                                      (OFF_RACE,))
    tables = lax.dynamic_update_slice(tables, gend_table.reshape(-1),
                                      (OFF_GEND,))
    tables = lax.dynamic_update_slice(tables, align_table.reshape(-1),
                                      (OFF_ALIGN,))
    return _hero(role.astype(jnp.int32), race.astype(jnp.int32),
                 gend.astype(jnp.int32), align.astype(jnp.int32), tables)


# speedup vs baseline: 1.4603x; 1.4603x over previous
import functools

import jax
import jax.numpy as jnp
from jax import lax
from jax.experimental import pallas as pl
from jax.experimental.pallas import tpu as pltpu, tpu_sc as plsc

ROLE_CAD, ROLE_EMB = 13, 8
RACE_CAD, RACE_EMB = 5, 4
GEND_CAD, GEND_EMB = 3, 2
ALIGN_CAD, ALIGN_EMB = 3, 2
B = 16384
D = ROLE_EMB + RACE_EMB + GEND_EMB + ALIGN_EMB

NC, NS, L = 2, 16, 16
NW = NC * NS
BPW = B // NW
CHUNKS = BPW // L

OFF_ROLE = 0
OFF_RACE = OFF_ROLE + ROLE_CAD * ROLE_EMB
OFF_GEND = OFF_RACE + RACE_CAD * RACE_EMB
OFF_ALIGN = OFF_GEND + GEND_CAD * GEND_EMB
FLAT_LEN = OFF_ALIGN + ALIGN_CAD * ALIGN_EMB


def _hero_body(role_h, race_h, gend_h, align_h, tab_h, out_h,
               flat_v, ri_v, ci_v, gi_v, ai_v, out_v):
    wid = lax.axis_index("s") * NC + lax.axis_index("c")
    base = wid * BPW

    pltpu.sync_copy(tab_h, flat_v)
    pltpu.sync_copy(role_h.at[pl.ds(base, BPW)], ri_v)
    pltpu.sync_copy(race_h.at[pl.ds(base, BPW)], ci_v)
    pltpu.sync_copy(gend_h.at[pl.ds(base, BPW)], gi_v)
    pltpu.sync_copy(align_h.at[pl.ds(base, BPW)], ai_v)

    lane = lax.iota(jnp.int32, L)

    @plsc.parallel_loop(0, CHUNKS, step=1, unroll=4)
    def chunk_body(k):
        e0 = k * L
        r = ri_v[pl.ds(e0, L)]
        c = ci_v[pl.ds(e0, L)]
        g = gi_v[pl.ds(e0, L)]
        a = ai_v[pl.ds(e0, L)]
        a = jnp.minimum(jnp.maximum(a + 1, 0), ALIGN_CAD - 1)
        for col in range(D):
            if col < ROLE_EMB:
                off = OFF_ROLE + r * ROLE_EMB + col
            elif col < ROLE_EMB + RACE_EMB:
                off = OFF_RACE + c * RACE_EMB + (col - ROLE_EMB)
            elif col < ROLE_EMB + RACE_EMB + GEND_EMB:
                off = OFF_GEND + g * GEND_EMB + (col - ROLE_EMB - RACE_EMB)
            else:
                off = (OFF_ALIGN + a * ALIGN_EMB
                       + (col - ROLE_EMB - RACE_EMB - GEND_EMB))
            vals = plsc.load_gather(flat_v, [off])
            out_v[col, pl.ds(e0, L)] = vals

    pltpu.sync_copy(out_v, out_h.at[:, pl.ds(base, BPW)])


_hero = functools.partial(
    pl.kernel,
    out_type=jax.ShapeDtypeStruct((D, B), jnp.float32),
    mesh=plsc.VectorSubcoreMesh(core_axis_name="c", subcore_axis_name="s"),
    compiler_params=pltpu.CompilerParams(needs_layout_passes=False),
    scratch_types=[
        pltpu.VMEM((FLAT_LEN,), jnp.float32),
        pltpu.VMEM((BPW,), jnp.int32),
        pltpu.VMEM((BPW,), jnp.int32),
        pltpu.VMEM((BPW,), jnp.int32),
        pltpu.VMEM((BPW,), jnp.int32),
        pltpu.VMEM((D, BPW), jnp.float32),
    ],
)(_hero_body)


def kernel(role, race, gend, align, role_table, race_table, gend_table,
           align_table):
    tables = jnp.zeros((FLAT_LEN,), jnp.float32)
    tables = lax.dynamic_update_slice(tables, role_table.reshape(-1),
                                      (OFF_ROLE,))
    tables = lax.dynamic_update_slice(tables, race_table.reshape(-1),
                                      (OFF_RACE,))
    tables = lax.dynamic_update_slice(tables, gend_table.reshape(-1),
                                      (OFF_GEND,))
    tables = lax.dynamic_update_slice(tables, align_table.reshape(-1),
                                      (OFF_ALIGN,))
    out_cm = _hero(role.astype(jnp.int32), race.astype(jnp.int32),
                   gend.astype(jnp.int32), align.astype(jnp.int32), tables)
    return out_cm.T
